# baseline (speedup 1.0000x reference)
import functools

import jax
import jax.numpy as jnp
from jax import lax
from jax.experimental import pallas as pl
from jax.experimental.pallas import tpu as pltpu
from jax.experimental.pallas import tpu_sc as plsc

N = 10000
D = 128
R = 4
E = 80000

NC = 2
NS = 16
CHUNK = 128
NCHUNK = E // CHUNK
CPT = 40
CPT_LAST = NCHUNK - (NS - 1) * CPT
N_ACC = N + 8
ROWS_PER_TILE = N // NS
CNT_SPLIT = 632


def _wx_body(x_ref, nw_ref, o_ref):
    o_ref[...] = x_ref[...] * nw_ref[...][..., None]


def _wx(x, nw):
    out = pl.pallas_call(
        _wx_body,
        out_shape=jax.ShapeDtypeStruct((N // 8, 8, D), jnp.float32),
    )(x.reshape(N // 8, 8, D), nw.reshape(N // 8, 8))
    return out.reshape(N, D)


def _seg_body(wx_hbm, src_hbm, tgt_hbm, a_hbm, cnt_hbm,
              srcv, tgtv, rows0, rows1, ones, zrows, zcnt, cstage, acc, cnt,
              sem0, sem1):
    c = lax.axis_index("c")
    s = lax.axis_index("s")

    def _init_row(i, _):
        for j in range(D // 16):
            zrows[i, pl.ds(j * 16, 16)] = jnp.zeros((16,), jnp.float32)
        return 0
    lax.fori_loop(0, 25, _init_row, 0)
    for j in range(CHUNK // 16):
        ones[pl.ds(j * 16, 16)] = jnp.ones((16,), jnp.float32)
    def _init_zc(i, _):
        zcnt[pl.ds(i * 16, 16)] = jnp.zeros((16,), jnp.float32)
        return 0
    lax.fori_loop(0, CNT_SPLIT // 16, _init_zc, 0)

    for rel in range(2):
        r = rel * 2 + c

        for j in range(ROWS_PER_TILE // 25):
            pltpu.sync_copy(zrows, acc.at[pl.ds(s * ROWS_PER_TILE + j * 25, 25)])
        base_c = s * CNT_SPLIT
        @pl.when(s < NS - 1)
        def _():
            pltpu.sync_copy(zcnt, cnt.at[pl.ds(base_c, CNT_SPLIT)])
        @pl.when(s == NS - 1)
        def _():
            pltpu.sync_copy(zcnt.at[pl.ds(0, N - (NS - 1) * CNT_SPLIT)],
                            cnt.at[pl.ds((NS - 1) * CNT_SPLIT, N - (NS - 1) * CNT_SPLIT)])
        plsc.subcore_barrier()

        @pl.when(s < NS - 1)
        def _():
            pltpu.sync_copy(src_hbm.at[r, pl.ds(s * CPT, CPT)], srcv)
            pltpu.sync_copy(tgt_hbm.at[r, pl.ds(s * CPT, CPT)], tgtv)
        @pl.when(s == NS - 1)
        def _():
            pltpu.sync_copy(src_hbm.at[r, pl.ds((NS - 1) * CPT, CPT_LAST)],
                            srcv.at[pl.ds(0, CPT_LAST)])
            pltpu.sync_copy(tgt_hbm.at[r, pl.ds((NS - 1) * CPT, CPT_LAST)],
                            tgtv.at[pl.ds(0, CPT_LAST)])
        nch = jnp.where(s == NS - 1, CPT_LAST, CPT)

        def _scat(i, buf):
            pltpu.sync_copy(buf, acc.at[tgtv.at[i]], add=True)
            pltpu.sync_copy(ones, cnt.at[tgtv.at[i]], add=True)

        pltpu.async_copy(wx_hbm.at[srcv.at[0]], rows0, sem0)
        def _pipe(ii, _):
            k0 = ii * 2
            pltpu.async_copy(wx_hbm.at[srcv.at[k0 + 1]], rows1, sem1)
            pltpu.make_async_copy(wx_hbm.at[srcv.at[k0]], rows0, sem0).wait()
            _scat(k0, rows0)
            @pl.when(k0 + 2 < nch)
            def _():
                pltpu.async_copy(wx_hbm.at[srcv.at[k0 + 2]], rows0, sem0)
            pltpu.make_async_copy(wx_hbm.at[srcv.at[k0 + 1]], rows1, sem1).wait()
            _scat(k0 + 1, rows1)
            return 0
        lax.fori_loop(0, nch // 2, _pipe, 0)
        @pl.when(s == NS - 1)
        def _():
            pltpu.make_async_copy(wx_hbm.at[srcv.at[CPT_LAST - 1]], rows0,
                                  sem0).wait()
            _scat(CPT_LAST - 1, rows0)

        plsc.subcore_barrier()

        @pl.when(s < NS - 1)
        def _():
            pltpu.sync_copy(acc.at[pl.ds(base_c, CNT_SPLIT)],
                            a_hbm.at[r, pl.ds(base_c, CNT_SPLIT)])
        @pl.when(s == NS - 1)
        def _():
            pltpu.sync_copy(
                acc.at[pl.ds((NS - 1) * CNT_SPLIT, N - (NS - 1) * CNT_SPLIT)],
                a_hbm.at[r, pl.ds((NS - 1) * CNT_SPLIT, N - (NS - 1) * CNT_SPLIT)])
        coff = pl.multiple_of(r * N + base_c, 8)
        @pl.when(s < NS - 1)
        def _():
            pltpu.sync_copy(cnt.at[pl.ds(base_c, CNT_SPLIT)], cstage)
            pltpu.sync_copy(cstage, cnt_hbm.at[pl.ds(coff, CNT_SPLIT)])
        @pl.when(s == NS - 1)
        def _():
            tail = N - (NS - 1) * CNT_SPLIT
            pltpu.sync_copy(cnt.at[pl.ds((NS - 1) * CNT_SPLIT, tail)],
                            cstage.at[pl.ds(0, tail)])
            pltpu.sync_copy(cstage.at[pl.ds(0, tail)],
                            cnt_hbm.at[pl.ds(coff, tail)])
        plsc.subcore_barrier()


_seg = functools.partial(
    pl.kernel,
    _seg_body,
    out_type=[
        jax.ShapeDtypeStruct((R, N, D), jnp.float32),
        jax.ShapeDtypeStruct((R * N,), jnp.float32),
    ],
    mesh=plsc.VectorSubcoreMesh(
        core_axis_name="c", subcore_axis_name="s",
        num_cores=NC, num_subcores=NS),
    scratch_types=[
        pltpu.VMEM((CPT, CHUNK), jnp.int32),
        pltpu.VMEM((CPT, CHUNK), jnp.int32),
        pltpu.VMEM((CHUNK, D), jnp.float32),
        pltpu.VMEM((CHUNK, D), jnp.float32),
        pltpu.VMEM((CHUNK,), jnp.float32),
        pltpu.VMEM((25, D), jnp.float32),
        pltpu.VMEM((CNT_SPLIT,), jnp.float32),
        pltpu.VMEM((CNT_SPLIT,), jnp.float32),
        pltpu.VMEM_SHARED((N_ACC, D), jnp.float32),
        pltpu.VMEM_SHARED((N_ACC,), jnp.float32),
        pltpu.SemaphoreType.DMA,
        pltpu.SemaphoreType.DMA,
    ],
)()


BLK = 1000


def _fin_body(x_ref, a_ref, cnt_ref, wrel_ref, wself_ref, b_ref, g_ref, be_ref,
              o_ref):
    x = x_ref[...]
    out = lax.dot_general(x, wself_ref[...], (((1,), (1,)), ((), ())),
                          preferred_element_type=jnp.float32) + b_ref[...]
    inv = 1.0 / jnp.clip(cnt_ref[...], 1.0, None)
    for r in range(R):
        ar = a_ref[r] * inv[:, r:r + 1]
        out = out + lax.dot_general(ar, wrel_ref[r], (((1,), (1,)), ((), ())),
                                    preferred_element_type=jnp.float32)
    h = jnp.maximum(out, 0.0)
    mean = jnp.mean(h, axis=1, keepdims=True)
    cen = h - mean
    var = jnp.mean(cen * cen, axis=1, keepdims=True)
    hn = cen * lax.rsqrt(var + 1e-5)
    o_ref[...] = hn * g_ref[...] + be_ref[...]


def _final(x, a, cnt_t, w_rel, w_self, b2, g2, be2):
    grid = N // BLK
    return pl.pallas_call(
        _fin_body,
        grid=(grid,),
        in_specs=[
            pl.BlockSpec((BLK, D), lambda i: (i, 0)),
            pl.BlockSpec((R, BLK, D), lambda i: (0, i, 0)),
            pl.BlockSpec((BLK, R), lambda i: (i, 0)),
            pl.BlockSpec((R, D, D), lambda i: (0, 0, 0)),
            pl.BlockSpec((D, D), lambda i: (0, 0)),
            pl.BlockSpec((1, D), lambda i: (0, 0)),
            pl.BlockSpec((1, D), lambda i: (0, 0)),
            pl.BlockSpec((1, D), lambda i: (0, 0)),
        ],
        out_specs=pl.BlockSpec((BLK, D), lambda i: (i, 0)),
        out_shape=jax.ShapeDtypeStruct((N, D), jnp.float32),
    )(x, a, cnt_t, w_rel, w_self, b2, g2, be2)


def kernel(x, node_weights, edge_index_r0, edge_index_r1, edge_index_r2,
           edge_index_r3, W_rel, W_self, b_self, ln_gamma, ln_beta):
    edges = jnp.stack([edge_index_r0, edge_index_r1, edge_index_r2,
                       edge_index_r3]).astype(jnp.int32)
    src = edges[:, 0, :].reshape(R, NCHUNK, CHUNK)
    tgt = edges[:, 1, :].reshape(R, NCHUNK, CHUNK)

    wx = _wx(x, node_weights)
    a, cnt = _seg(wx, src, tgt)
    cnt = cnt.reshape(R, N)
    return _final(x, a, cnt.T, W_rel, W_self, b_self.reshape(1, D),
                  ln_gamma.reshape(1, D), ln_beta.reshape(1, D))


# speedup vs baseline: 1.0217x; 1.0217x over previous
import functools

import jax
import jax.numpy as jnp
from jax import lax
from jax.experimental import pallas as pl
from jax.experimental.pallas import tpu as pltpu
from jax.experimental.pallas import tpu_sc as plsc

N = 10000
D = 128
R = 4
E = 80000

NC = 2
NS = 16
CHUNK = 128
CPT = 40
E_PAD = NS * CPT * CHUNK
N_ACC = N + 8
ROWS_PER_TILE = N // NS
CNT_SPLIT = 632


def _wx_body(x_ref, nw_ref, o_ref):
    o_ref[...] = x_ref[...] * nw_ref[...][..., None]


def _wx(x, nw):
    out = pl.pallas_call(
        _wx_body,
        out_shape=jax.ShapeDtypeStruct((N // 8, 8, D), jnp.float32),
    )(x.reshape(N // 8, 8, D), nw.reshape(N // 8, 8))
    return out.reshape(N, D)


def _seg_body(wx_hbm, src_hbm, tgt_hbm, a_hbm, cnt_hbm,
              srcv, tgtv, rows0, rows1, ones, zrows, zcnt, cstage, acc, cnt,
              sem0, sem1):
    c = lax.axis_index("c")
    s = lax.axis_index("s")

    def _init_row(i, _):
        for j in range(D // 16):
            zrows[i, pl.ds(j * 16, 16)] = jnp.zeros((16,), jnp.float32)
        return 0
    lax.fori_loop(0, 25, _init_row, 0)
    for j in range(CHUNK // 16):
        ones[pl.ds(j * 16, 16)] = jnp.ones((16,), jnp.float32)
    def _init_zc(i, _):
        zcnt[pl.ds(i * 16, 16)] = jnp.zeros((16,), jnp.float32)
        return 0
    lax.fori_loop(0, CNT_SPLIT // 16, _init_zc, 0)

    for rel in range(2):
        r = rel * 2 + c

        for j in range(ROWS_PER_TILE // 25):
            pltpu.sync_copy(zrows, acc.at[pl.ds(s * ROWS_PER_TILE + j * 25, 25)])
        base_c = s * CNT_SPLIT
        @pl.when(s < NS - 1)
        def _():
            pltpu.sync_copy(zcnt, cnt.at[pl.ds(base_c, CNT_SPLIT)])
        @pl.when(s == NS - 1)
        def _():
            pltpu.sync_copy(zcnt.at[pl.ds(0, N - (NS - 1) * CNT_SPLIT)],
                            cnt.at[pl.ds((NS - 1) * CNT_SPLIT, N - (NS - 1) * CNT_SPLIT)])
        plsc.subcore_barrier()

        pltpu.sync_copy(src_hbm.at[r, s], srcv)
        pltpu.sync_copy(tgt_hbm.at[r, s], tgtv)

        def _scat(i, buf):
            pltpu.sync_copy(buf, acc.at[tgtv.at[i]], add=True)
            pltpu.sync_copy(ones, cnt.at[tgtv.at[i]], add=True)

        pltpu.async_copy(wx_hbm.at[srcv.at[0]], rows0, sem0)
        def _pipe(ii, _):
            k0 = ii * 2
            pltpu.async_copy(wx_hbm.at[srcv.at[k0 + 1]], rows1, sem1)
            pltpu.make_async_copy(wx_hbm.at[srcv.at[k0]], rows0, sem0).wait()
            _scat(k0, rows0)
            @pl.when(k0 + 2 < CPT)
            def _():
                pltpu.async_copy(wx_hbm.at[srcv.at[k0 + 2]], rows0, sem0)
            pltpu.make_async_copy(wx_hbm.at[srcv.at[k0 + 1]], rows1, sem1).wait()
            _scat(k0 + 1, rows1)
            return 0
        lax.fori_loop(0, CPT // 2, _pipe, 0)

        plsc.subcore_barrier()

        @pl.when(s < NS - 1)
        def _():
            pltpu.sync_copy(acc.at[pl.ds(base_c, CNT_SPLIT)],
                            a_hbm.at[r, pl.ds(base_c, CNT_SPLIT)])
        @pl.when(s == NS - 1)
        def _():
            pltpu.sync_copy(
                acc.at[pl.ds((NS - 1) * CNT_SPLIT, N - (NS - 1) * CNT_SPLIT)],
                a_hbm.at[r, pl.ds((NS - 1) * CNT_SPLIT, N - (NS - 1) * CNT_SPLIT)])
        coff = pl.multiple_of(r * N + base_c, 8)
        @pl.when(s < NS - 1)
        def _():
            pltpu.sync_copy(cnt.at[pl.ds(base_c, CNT_SPLIT)], cstage)
            pltpu.sync_copy(cstage, cnt_hbm.at[pl.ds(coff, CNT_SPLIT)])
        @pl.when(s == NS - 1)
        def _():
            tail = N - (NS - 1) * CNT_SPLIT
            pltpu.sync_copy(cnt.at[pl.ds((NS - 1) * CNT_SPLIT, tail)],
                            cstage.at[pl.ds(0, tail)])
            pltpu.sync_copy(cstage.at[pl.ds(0, tail)],
                            cnt_hbm.at[pl.ds(coff, tail)])
        plsc.subcore_barrier()


_seg = functools.partial(
    pl.kernel,
    _seg_body,
    out_type=[
        jax.ShapeDtypeStruct((R, N, D), jnp.float32),
        jax.ShapeDtypeStruct((R * N,), jnp.float32),
    ],
    mesh=plsc.VectorSubcoreMesh(
        core_axis_name="c", subcore_axis_name="s",
        num_cores=NC, num_subcores=NS),
    scratch_types=[
        pltpu.VMEM((CPT, CHUNK), jnp.int32),
        pltpu.VMEM((CPT, CHUNK), jnp.int32),
        pltpu.VMEM((CHUNK, D), jnp.float32),
        pltpu.VMEM((CHUNK, D), jnp.float32),
        pltpu.VMEM((CHUNK,), jnp.float32),
        pltpu.VMEM((25, D), jnp.float32),
        pltpu.VMEM((CNT_SPLIT,), jnp.float32),
        pltpu.VMEM((CNT_SPLIT,), jnp.float32),
        pltpu.VMEM_SHARED((N_ACC, D), jnp.float32),
        pltpu.VMEM_SHARED((N_ACC,), jnp.float32),
        pltpu.SemaphoreType.DMA,
        pltpu.SemaphoreType.DMA,
    ],
)()


BLK = 1000


def _fin_body(x_ref, a_ref, cnt_ref, wrel_ref, wself_ref, b_ref, g_ref, be_ref,
              o_ref):
    x = x_ref[...]
    out = lax.dot_general(x, wself_ref[...], (((1,), (1,)), ((), ())),
                          preferred_element_type=jnp.float32) + b_ref[...]
    inv = 1.0 / jnp.clip(cnt_ref[...], 1.0, None)
    for r in range(R):
        ar = a_ref[r] * inv[:, r:r + 1]
        out = out + lax.dot_general(ar, wrel_ref[r], (((1,), (1,)), ((), ())),
                                    preferred_element_type=jnp.float32)
    h = jnp.maximum(out, 0.0)
    mean = jnp.mean(h, axis=1, keepdims=True)
    cen = h - mean
    var = jnp.mean(cen * cen, axis=1, keepdims=True)
    hn = cen * lax.rsqrt(var + 1e-5)
    o_ref[...] = hn * g_ref[...] + be_ref[...]


def _final(x, a, cnt_t, w_rel, w_self, b2, g2, be2):
    grid = N // BLK
    return pl.pallas_call(
        _fin_body,
        grid=(grid,),
        in_specs=[
            pl.BlockSpec((BLK, D), lambda i: (i, 0)),
            pl.BlockSpec((R, BLK, D), lambda i: (0, i, 0)),
            pl.BlockSpec((BLK, R), lambda i: (i, 0)),
            pl.BlockSpec((R, D, D), lambda i: (0, 0, 0)),
            pl.BlockSpec((D, D), lambda i: (0, 0)),
            pl.BlockSpec((1, D), lambda i: (0, 0)),
            pl.BlockSpec((1, D), lambda i: (0, 0)),
            pl.BlockSpec((1, D), lambda i: (0, 0)),
        ],
        out_specs=pl.BlockSpec((BLK, D), lambda i: (i, 0)),
        out_shape=jax.ShapeDtypeStruct((N, D), jnp.float32),
    )(x, a, cnt_t, w_rel, w_self, b2, g2, be2)


def kernel(x, node_weights, edge_index_r0, edge_index_r1, edge_index_r2,
           edge_index_r3, W_rel, W_self, b_self, ln_gamma, ln_beta):
    edges = jnp.stack([edge_index_r0, edge_index_r1, edge_index_r2,
                       edge_index_r3]).astype(jnp.int32)
    npad = E_PAD - E
    pad_src = (jnp.arange(npad, dtype=jnp.int32) * 613) % N
    pad_tgt = N + (jnp.arange(npad, dtype=jnp.int32) % (N_ACC - N))
    src = jnp.concatenate(
        [edges[:, 0, :], jnp.broadcast_to(pad_src, (R, npad))], axis=1
    ).reshape(R, NS, CPT, CHUNK)
    tgt = jnp.concatenate(
        [edges[:, 1, :], jnp.broadcast_to(pad_tgt, (R, npad))], axis=1
    ).reshape(R, NS, CPT, CHUNK)

    wx = _wx(x, node_weights)
    a, cnt = _seg(wx, src, tgt)
    cnt = cnt.reshape(R, N)
    return _final(x, a, cnt.T, W_rel, W_self, b_self.reshape(1, D),
                  ln_gamma.reshape(1, D), ln_beta.reshape(1, D))


# speedup vs baseline: 1.0381x; 1.0161x over previous
import functools

import jax
import jax.numpy as jnp
from jax import lax
from jax.experimental import pallas as pl
from jax.experimental.pallas import tpu as pltpu
from jax.experimental.pallas import tpu_sc as plsc

N = 10000
D = 128
R = 4
E = 80000

NC = 2
NS = 16
CHUNK = 128
CPT = 40
E_PAD = NS * CPT * CHUNK
N_ACC = N + 8
ROWS_PER_TILE = N // NS
CNT_SPLIT = 632


def _wx_body(x_ref, nw_ref, o_ref):
    o_ref[...] = x_ref[...] * nw_ref[...][..., None]


def _wx(x, nw):
    out = pl.pallas_call(
        _wx_body,
        out_shape=jax.ShapeDtypeStruct((N // 8, 8, D), jnp.float32),
    )(x.reshape(N // 8, 8, D), nw.reshape(N // 8, 8))
    return out.reshape(N, D)


def _seg_body(wx_hbm, src_hbm, tgt_hbm, a_hbm, cnt_hbm,
              srcv, tgtv, rows0, rows1, ones, zrows, zcnt, cstage, acc, cnt,
              sem0, sem1):
    c = lax.axis_index("c")
    s = lax.axis_index("s")

    def _init_row(i, _):
        for j in range(D // 16):
            zrows[i, pl.ds(j * 16, 16)] = jnp.zeros((16,), jnp.float32)
        return 0
    lax.fori_loop(0, 25, _init_row, 0)
    for j in range(CHUNK // 16):
        ones[pl.ds(j * 16, 16)] = jnp.ones((16,), jnp.float32)
    def _init_zc(i, _):
        zcnt[pl.ds(i * 16, 16)] = jnp.zeros((16,), jnp.float32)
        return 0
    lax.fori_loop(0, CNT_SPLIT // 16, _init_zc, 0)

    for rel in range(2):
        r = rel * 2 + c

        for j in range(ROWS_PER_TILE // 25):
            pltpu.sync_copy(zrows, acc.at[pl.ds(s * ROWS_PER_TILE + j * 25, 25)])
        base_c = s * CNT_SPLIT
        @pl.when(s < NS - 1)
        def _():
            pltpu.sync_copy(zcnt, cnt.at[pl.ds(base_c, CNT_SPLIT)])
        @pl.when(s == NS - 1)
        def _():
            pltpu.sync_copy(zcnt.at[pl.ds(0, N - (NS - 1) * CNT_SPLIT)],
                            cnt.at[pl.ds((NS - 1) * CNT_SPLIT, N - (NS - 1) * CNT_SPLIT)])
        plsc.subcore_barrier()

        pltpu.sync_copy(src_hbm.at[r, s], srcv)
        pltpu.sync_copy(tgt_hbm.at[r, s], tgtv)

        def _scat(i, buf):
            pltpu.sync_copy(buf, acc.at[tgtv.at[i]], add=True)
            pltpu.sync_copy(ones, cnt.at[tgtv.at[i]], add=True)

        pltpu.async_copy(wx_hbm.at[srcv.at[0]], rows0, sem0)
        def _pipe(ii, _):
            k0 = ii * 2
            pltpu.async_copy(wx_hbm.at[srcv.at[k0 + 1]], rows1, sem1)
            pltpu.make_async_copy(wx_hbm.at[srcv.at[k0]], rows0, sem0).wait()
            _scat(k0, rows0)
            @pl.when(k0 + 2 < CPT)
            def _():
                pltpu.async_copy(wx_hbm.at[srcv.at[k0 + 2]], rows0, sem0)
            pltpu.make_async_copy(wx_hbm.at[srcv.at[k0 + 1]], rows1, sem1).wait()
            _scat(k0 + 1, rows1)
            return 0
        lax.fori_loop(0, CPT // 2, _pipe, 0)

        plsc.subcore_barrier()

        @pl.when(s < NS - 1)
        def _():
            pltpu.sync_copy(acc.at[pl.ds(base_c, CNT_SPLIT)],
                            a_hbm.at[r, pl.ds(base_c, CNT_SPLIT)])
        @pl.when(s == NS - 1)
        def _():
            pltpu.sync_copy(
                acc.at[pl.ds((NS - 1) * CNT_SPLIT, N - (NS - 1) * CNT_SPLIT)],
                a_hbm.at[r, pl.ds((NS - 1) * CNT_SPLIT, N - (NS - 1) * CNT_SPLIT)])
        coff = pl.multiple_of(r * N + base_c, 8)
        @pl.when(s < NS - 1)
        def _():
            pltpu.sync_copy(cnt.at[pl.ds(base_c, CNT_SPLIT)], cstage)
            pltpu.sync_copy(cstage, cnt_hbm.at[pl.ds(coff, CNT_SPLIT)])
        @pl.when(s == NS - 1)
        def _():
            tail = N - (NS - 1) * CNT_SPLIT
            pltpu.sync_copy(cnt.at[pl.ds((NS - 1) * CNT_SPLIT, tail)],
                            cstage.at[pl.ds(0, tail)])
            pltpu.sync_copy(cstage.at[pl.ds(0, tail)],
                            cnt_hbm.at[pl.ds(coff, tail)])
        plsc.subcore_barrier()


_seg = functools.partial(
    pl.kernel,
    _seg_body,
    out_type=[
        jax.ShapeDtypeStruct((R, N, D), jnp.float32),
        jax.ShapeDtypeStruct((R * N,), jnp.float32),
    ],
    mesh=plsc.VectorSubcoreMesh(
        core_axis_name="c", subcore_axis_name="s",
        num_cores=NC, num_subcores=NS),
    scratch_types=[
        pltpu.VMEM((CPT, CHUNK), jnp.int32),
        pltpu.VMEM((CPT, CHUNK), jnp.int32),
        pltpu.VMEM((CHUNK, D), jnp.float32),
        pltpu.VMEM((CHUNK, D), jnp.float32),
        pltpu.VMEM((CHUNK,), jnp.float32),
        pltpu.VMEM((25, D), jnp.float32),
        pltpu.VMEM((CNT_SPLIT,), jnp.float32),
        pltpu.VMEM((CNT_SPLIT,), jnp.float32),
        pltpu.VMEM_SHARED((N_ACC, D), jnp.float32),
        pltpu.VMEM_SHARED((N_ACC,), jnp.float32),
        pltpu.SemaphoreType.DMA,
        pltpu.SemaphoreType.DMA,
    ],
)()


BLK = 2000


def _fin_body(x_ref, a_ref, cnt_ref, wrel_ref, wself_ref, b_ref, g_ref, be_ref,
              o_ref):
    x = x_ref[...]
    out = lax.dot_general(x, wself_ref[...], (((1,), (1,)), ((), ())),
                          preferred_element_type=jnp.float32) + b_ref[...]
    inv = 1.0 / jnp.clip(cnt_ref[...], 1.0, None)
    for r in range(R):
        ar = a_ref[r] * inv[:, r:r + 1]
        out = out + lax.dot_general(ar, wrel_ref[r], (((1,), (1,)), ((), ())),
                                    preferred_element_type=jnp.float32)
    h = jnp.maximum(out, 0.0)
    mean = jnp.mean(h, axis=1, keepdims=True)
    cen = h - mean
    var = jnp.mean(cen * cen, axis=1, keepdims=True)
    hn = cen * lax.rsqrt(var + 1e-5)
    o_ref[...] = hn * g_ref[...] + be_ref[...]


def _final(x, a, cnt_t, w_rel, w_self, b2, g2, be2):
    grid = N // BLK
    return pl.pallas_call(
        _fin_body,
        grid=(grid,),
        in_specs=[
            pl.BlockSpec((BLK, D), lambda i: (i, 0)),
            pl.BlockSpec((R, BLK, D), lambda i: (0, i, 0)),
            pl.BlockSpec((BLK, R), lambda i: (i, 0)),
            pl.BlockSpec((R, D, D), lambda i: (0, 0, 0)),
            pl.BlockSpec((D, D), lambda i: (0, 0)),
            pl.BlockSpec((1, D), lambda i: (0, 0)),
            pl.BlockSpec((1, D), lambda i: (0, 0)),
            pl.BlockSpec((1, D), lambda i: (0, 0)),
        ],
        out_specs=pl.BlockSpec((BLK, D), lambda i: (i, 0)),
        out_shape=jax.ShapeDtypeStruct((N, D), jnp.float32),
    )(x, a, cnt_t, w_rel, w_self, b2, g2, be2)


def kernel(x, node_weights, edge_index_r0, edge_index_r1, edge_index_r2,
           edge_index_r3, W_rel, W_self, b_self, ln_gamma, ln_beta):
    edges = jnp.stack([edge_index_r0, edge_index_r1, edge_index_r2,
                       edge_index_r3]).astype(jnp.int32)
    npad = E_PAD - E
    pad_src = (jnp.arange(npad, dtype=jnp.int32) * 613) % N
    pad_tgt = N + (jnp.arange(npad, dtype=jnp.int32) % (N_ACC - N))
    src = jnp.concatenate(
        [edges[:, 0, :], jnp.broadcast_to(pad_src, (R, npad))], axis=1
    ).reshape(R, NS, CPT, CHUNK)
    tgt = jnp.concatenate(
        [edges[:, 1, :], jnp.broadcast_to(pad_tgt, (R, npad))], axis=1
    ).reshape(R, NS, CPT, CHUNK)

    wx = _wx(x, node_weights)
    a, cnt = _seg(wx, src, tgt)
    cnt = cnt.reshape(R, N)
    return _final(x, a, cnt.T, W_rel, W_self, b_self.reshape(1, D),
                  ln_gamma.reshape(1, D), ln_beta.reshape(1, D))
